# baseline (speedup 1.0000x reference)
import math

import jax
import jax.numpy as jnp
from jax import lax
from jax.experimental import pallas as pl
from jax.experimental.pallas import tpu as pltpu
from jax.experimental.pallas import tpu_sc as plsc

B, L, C = 4096, 50, 128
SCALE = math.sqrt(C)
NC, NS = 2, 16
NW = NC * NS
BPW = B // NW


def _sc_body(ph_tbl, f2_tbl, ph_idx, f2_idx, a1_in, out_hbm,
             idx_ph_v, idx_f2_v, a1_v,
             ph_rows0, ph_rows1, f2_rows0, f2_rows1,
             ph_sec0, ph_sec1, f2_sec0, f2_sec1, a1_sec0, a1_sec1,
             gsem0, gsem1, osem0, osem1):
    wid = lax.axis_index("s") * NC + lax.axis_index("c")
    base = wid * BPW
    iota = lax.iota(jnp.int32, 16)
    ph_rows = (ph_rows0, ph_rows1)
    f2_rows = (f2_rows0, f2_rows1)
    secs = (((ph_sec0, f2_sec0, a1_sec0)), (ph_sec1, f2_sec1, a1_sec1))
    gsem = (gsem0, gsem1)
    osem = (osem0, osem1)
    bases = [(iota + c0) * L for c0 in range(0, C, 16)]

    pltpu.sync_copy(ph_idx.at[pl.ds(base, BPW)], idx_ph_v)
    pltpu.sync_copy(f2_idx.at[pl.ds(base, BPW)], idx_f2_v)
    pltpu.sync_copy(a1_in.at[pl.ds(base * L, BPW * L)], a1_v)

    def start_gathers(bl, p):
        pltpu.make_async_copy(ph_tbl.at[idx_ph_v.at[bl]], ph_rows[p],
                              gsem[p]).start()
        pltpu.make_async_copy(f2_tbl.at[idx_f2_v.at[bl]], f2_rows[p],
                              gsem[p]).start()

    def wait_gathers(bl, p):
        pltpu.make_async_copy(ph_tbl.at[idx_ph_v.at[bl]], ph_rows[p],
                              gsem[p]).wait()
        pltpu.make_async_copy(f2_tbl.at[idx_f2_v.at[bl]], f2_rows[p],
                              gsem[p]).wait()

    def start_out(b, p):
        for si in range(3):
            pltpu.make_async_copy(secs[p][si], out_hbm.at[b * 3 + si],
                                  osem[p]).start()

    def wait_out(b, p):
        for si in range(3):
            pltpu.make_async_copy(secs[p][si], out_hbm.at[b * 3 + si],
                                  osem[p]).wait()

    start_gathers(0, 0)

    def pair_body(k, carry):
        for p in range(2):
            bl = k * 2 + p
            b = base + bl

            @pl.when(bl + 1 < BPW)
            def _():
                start_gathers(bl + 1, 1 - p)

            wait_gathers(bl, p)

            @pl.when(k > 0)
            def _():
                wait_out(b, p)

            def l_body(l, carry2):
                av = plsc.load_gather(a1_v, [jnp.full((16,), bl * L,
                                                      jnp.int32) + l])
                for ci in range(8):
                    idx = bases[ci] + l
                    v = ph_rows[p][l, pl.ds(ci * 16, 16)] * SCALE
                    plsc.store_scatter(secs[p][0], [idx], v)
                    w = f2_rows[p][l, pl.ds(ci * 16, 16)] * SCALE
                    plsc.store_scatter(secs[p][1], [idx], w)
                    plsc.store_scatter(secs[p][2], [idx], av)
                return carry2

            lax.fori_loop(0, L, l_body, 0)

            start_out(b, p)
        return carry

    lax.fori_loop(0, BPW // 2, pair_body, 0)

    for p in range(2):
        wait_out(base, p)


def kernel(phoneme, a1, f2, phoneme_table, f2_table):
    mesh = plsc.VectorSubcoreMesh(core_axis_name="c", subcore_axis_name="s")
    f = pl.kernel(
        _sc_body,
        out_type=jax.ShapeDtypeStruct((B * 3, C * L), jnp.float32),
        mesh=mesh,
        compiler_params=pltpu.CompilerParams(needs_layout_passes=False),
        scratch_types=[
            pltpu.VMEM((BPW, L), jnp.int32),
            pltpu.VMEM((BPW, L), jnp.int32),
            pltpu.VMEM((BPW * L,), jnp.float32),
            pltpu.VMEM((L, C), jnp.float32),
            pltpu.VMEM((L, C), jnp.float32),
            pltpu.VMEM((L, C), jnp.float32),
            pltpu.VMEM((L, C), jnp.float32),
            pltpu.VMEM((C * L,), jnp.float32),
            pltpu.VMEM((C * L,), jnp.float32),
            pltpu.VMEM((C * L,), jnp.float32),
            pltpu.VMEM((C * L,), jnp.float32),
            pltpu.VMEM((C * L,), jnp.float32),
            pltpu.VMEM((C * L,), jnp.float32),
            pltpu.SemaphoreType.DMA,
            pltpu.SemaphoreType.DMA,
            pltpu.SemaphoreType.DMA,
            pltpu.SemaphoreType.DMA,
        ],
    )
    out = f(phoneme_table, f2_table, phoneme, f2, a1.reshape(B * L))
    return out.reshape(B, 3 * C, L)


# speedup vs baseline: 2.0465x; 2.0465x over previous
import math

import jax
import jax.numpy as jnp
from jax import lax
from jax.experimental import pallas as pl
from jax.experimental.pallas import tpu as pltpu
from jax.experimental.pallas import tpu_sc as plsc

B, L, C = 4096, 50, 128
SCALE = math.sqrt(C)
NC, NS = 2, 16
NW = NC * NS
BPW = B // NW


def _sc_body(ph_tbl, f2_tbl, ph_idx, f2_idx, a1_in, out_hbm,
             idx_ph_v, idx_f2_v, a1_v,
             ph_rows0, ph_rows1, f2_rows0, f2_rows1,
             out_buf0, out_buf1,
             gsem0, gsem1, osem0, osem1):
    wid = lax.axis_index("s") * NC + lax.axis_index("c")
    base = wid * BPW
    iota = lax.iota(jnp.int32, 16)
    ph_rows = (ph_rows0, ph_rows1)
    f2_rows = (f2_rows0, f2_rows1)
    out_buf = (out_buf0, out_buf1)
    gsem = (gsem0, gsem1)
    osem = (osem0, osem1)
    bases = [(iota + c0) * L for c0 in range(0, C, 16)]

    pltpu.sync_copy(ph_idx.at[pl.ds(base, BPW)], idx_ph_v)
    pltpu.sync_copy(f2_idx.at[pl.ds(base, BPW)], idx_f2_v)
    pltpu.sync_copy(a1_in.at[pl.ds(base * L, BPW * L)], a1_v)

    def start_gathers(bl, p):
        pltpu.make_async_copy(ph_tbl.at[idx_ph_v.at[bl]], ph_rows[p],
                              gsem[p]).start()
        pltpu.make_async_copy(f2_tbl.at[idx_f2_v.at[bl]], f2_rows[p],
                              gsem[p]).start()

    def wait_gathers(bl, p):
        pltpu.make_async_copy(ph_tbl.at[idx_ph_v.at[bl]], ph_rows[p],
                              gsem[p]).wait()
        pltpu.make_async_copy(f2_tbl.at[idx_f2_v.at[bl]], f2_rows[p],
                              gsem[p]).wait()

    def start_out(b, p):
        pltpu.make_async_copy(out_buf[p], out_hbm.at[b], osem[p]).start()

    def wait_out(b, p):
        pltpu.make_async_copy(out_buf[p], out_hbm.at[b], osem[p]).wait()

    start_gathers(0, 0)

    def pair_body(k, carry):
        for p in range(2):
            bl = k * 2 + p
            b = base + bl

            @pl.when(bl + 1 < BPW)
            def _():
                start_gathers(bl + 1, 1 - p)

            wait_gathers(bl, p)

            @pl.when(k > 0)
            def _():
                wait_out(b, p)

            @plsc.parallel_loop(0, L, unroll=2)
            def l_body(l):
                av = plsc.load_gather(a1_v, [jnp.full((16,), bl * L,
                                                      jnp.int32) + l])
                for ci in range(8):
                    idx = bases[ci] + l
                    v = ph_rows[p][l, pl.ds(ci * 16, 16)] * SCALE
                    plsc.store_scatter(out_buf[p], [idx], v)
                    w = f2_rows[p][l, pl.ds(ci * 16, 16)] * SCALE
                    plsc.store_scatter(out_buf[p], [idx + C * L], w)
                    plsc.store_scatter(out_buf[p], [idx + 2 * C * L], av)

            start_out(b, p)
        return carry

    lax.fori_loop(0, BPW // 2, pair_body, 0)

    for p in range(2):
        wait_out(base, p)


def kernel(phoneme, a1, f2, phoneme_table, f2_table):
    mesh = plsc.VectorSubcoreMesh(core_axis_name="c", subcore_axis_name="s")
    f = pl.kernel(
        _sc_body,
        out_type=jax.ShapeDtypeStruct((B, 3 * C * L), jnp.float32),
        mesh=mesh,
        compiler_params=pltpu.CompilerParams(needs_layout_passes=False),
        scratch_types=[
            pltpu.VMEM((BPW, L), jnp.int32),
            pltpu.VMEM((BPW, L), jnp.int32),
            pltpu.VMEM((BPW * L,), jnp.float32),
            pltpu.VMEM((L, C), jnp.float32),
            pltpu.VMEM((L, C), jnp.float32),
            pltpu.VMEM((L, C), jnp.float32),
            pltpu.VMEM((L, C), jnp.float32),
            pltpu.VMEM((3 * C * L,), jnp.float32),
            pltpu.VMEM((3 * C * L,), jnp.float32),
            pltpu.SemaphoreType.DMA,
            pltpu.SemaphoreType.DMA,
            pltpu.SemaphoreType.DMA,
            pltpu.SemaphoreType.DMA,
        ],
    )
    out = f(phoneme_table, f2_table, phoneme, f2, a1.reshape(B * L))
    return out.reshape(B, 3 * C, L)


# speedup vs baseline: 7.3742x; 3.6033x over previous
import math

import jax
import jax.numpy as jnp
from jax import lax
from jax.experimental import pallas as pl
from jax.experimental.pallas import tpu as pltpu
from jax.experimental.pallas import tpu_sc as plsc

B, L, C = 4096, 50, 128
SCALE = math.sqrt(C)
NC, NS = 2, 16
NW = NC * NS
BPW = B // NW


def _sc_body(ph_tbl, f2_tbl, ph_idx, f2_idx, a1_in, out_hbm,
             idx_ph_v, idx_f2_v, a1_v,
             ph_st0, ph_st1, f2_st0, f2_st1, a1_st0, a1_st1,
             gsem0, gsem1, osem0, osem1):
    wid = lax.axis_index("s") * NC + lax.axis_index("c")
    base = wid * BPW
    ph_st = (ph_st0, ph_st1)
    f2_st = (f2_st0, f2_st1)
    a1_st = (a1_st0, a1_st1)
    gsem = (gsem0, gsem1)
    osem = (osem0, osem1)

    pltpu.sync_copy(ph_idx.at[wid], idx_ph_v)
    pltpu.sync_copy(f2_idx.at[wid], idx_f2_v)
    pltpu.sync_copy(a1_in.at[wid], a1_v)

    def start_gathers(l, p):
        pltpu.make_async_copy(ph_tbl.at[idx_ph_v.at[l]],
                              ph_st[p], gsem[p]).start()
        pltpu.make_async_copy(f2_tbl.at[idx_f2_v.at[l]],
                              f2_st[p], gsem[p]).start()

    def wait_gathers(l, p):
        pltpu.make_async_copy(ph_tbl.at[idx_ph_v.at[l]],
                              ph_st[p], gsem[p]).wait()
        pltpu.make_async_copy(f2_tbl.at[idx_f2_v.at[l]],
                              f2_st[p], gsem[p]).wait()

    def start_out(l, p):
        pltpu.make_async_copy(
            ph_st[p], out_hbm.at[l, pl.ds(base, BPW), pl.ds(0, C)],
            osem[p]).start()
        pltpu.make_async_copy(
            f2_st[p], out_hbm.at[l, pl.ds(base, BPW), pl.ds(C, C)],
            osem[p]).start()
        pltpu.make_async_copy(
            a1_st[p], out_hbm.at[l, pl.ds(base, BPW), pl.ds(2 * C, C)],
            osem[p]).start()

    def wait_out(l, p):
        pltpu.make_async_copy(
            ph_st[p], out_hbm.at[l, pl.ds(base, BPW), pl.ds(0, C)],
            osem[p]).wait()
        pltpu.make_async_copy(
            f2_st[p], out_hbm.at[l, pl.ds(base, BPW), pl.ds(C, C)],
            osem[p]).wait()
        pltpu.make_async_copy(
            a1_st[p], out_hbm.at[l, pl.ds(base, BPW), pl.ds(2 * C, C)],
            osem[p]).wait()

    start_gathers(0, 0)

    def pair_body(k, carry):
        for p in range(2):
            l = k * 2 + p

            @pl.when(l >= 1)
            def _():
                wait_out(l, 1 - p)

            @pl.when(l + 1 < L)
            def _():
                start_gathers(l + 1, 1 - p)

            @plsc.parallel_loop(0, BPW, unroll=2)
            def i_body(i):
                av = plsc.load_gather(a1_v, [jnp.full((16,), 0, jnp.int32)
                                             + (l * BPW + i)])
                for ck in range(0, C, 16):
                    a1_st[p][i, pl.ds(ck, 16)] = av

            wait_gathers(l, p)
            start_out(l, p)
        return carry

    lax.fori_loop(0, L // 2, pair_body, 0)

    wait_out(0, 1)


def kernel(phoneme, a1, f2, phoneme_table, f2_table):
    pht = phoneme_table * SCALE
    f2t = f2_table * SCALE
    ph_w = phoneme.reshape(NW, BPW, L).transpose(0, 2, 1)
    f2_w = f2.reshape(NW, BPW, L).transpose(0, 2, 1)
    a1_w = a1.reshape(NW, BPW, L).transpose(0, 2, 1).reshape(NW, L * BPW)

    mesh = plsc.VectorSubcoreMesh(core_axis_name="c", subcore_axis_name="s")
    f = pl.kernel(
        _sc_body,
        out_type=jax.ShapeDtypeStruct((L, B, 3 * C), jnp.float32),
        mesh=mesh,
        compiler_params=pltpu.CompilerParams(needs_layout_passes=False),
        scratch_types=[
            pltpu.VMEM((L, BPW), jnp.int32),
            pltpu.VMEM((L, BPW), jnp.int32),
            pltpu.VMEM((L * BPW,), jnp.float32),
            pltpu.VMEM((BPW, C), jnp.float32),
            pltpu.VMEM((BPW, C), jnp.float32),
            pltpu.VMEM((BPW, C), jnp.float32),
            pltpu.VMEM((BPW, C), jnp.float32),
            pltpu.VMEM((BPW, C), jnp.float32),
            pltpu.VMEM((BPW, C), jnp.float32),
            pltpu.SemaphoreType.DMA,
            pltpu.SemaphoreType.DMA,
            pltpu.SemaphoreType.DMA,
            pltpu.SemaphoreType.DMA,
        ],
    )
    out = f(pht, f2t, ph_w, f2_w, a1_w)
    return jnp.transpose(out, (1, 2, 0))
